# Initial kernel scaffold; baseline (speedup 1.0000x reference)
import jax
import jax.numpy as jnp
from jax.experimental import pallas as pl


def kernel(inputs):
    raise NotImplementedError("write your pallas kernel here")


# speedup vs baseline: 5.6970x; 5.6970x over previous
import functools

import jax
import jax.numpy as jnp
from jax import lax
from jax.experimental import pallas as pl
from jax.experimental.pallas import tpu as pltpu
from jax.experimental.pallas import tpu_sc as plsc

ROWS = 64
N = 8192
K = 1639
L = 16
NV = N // L
UNROLL = 8

NUM_CORES = 2
NUM_SUBCORES = 16
NUM_WORKERS = NUM_CORES * NUM_SUBCORES
ROWS_PER_WORKER = ROWS // NUM_WORKERS

_SIGN_FIX = 0x7FFFFFFF
_INT_MIN = -(2**31)


def _threshold_vec(c):
    bits = jnp.where(c < 0, jnp.bitwise_xor(c, jnp.int32(_SIGN_FIX)), c)
    t = lax.bitcast_convert_type(bits, jnp.float32)
    return jnp.full((L,), t, dtype=jnp.float32)


def _count_ge(d_ref, c):
    tvec = _threshold_vec(c)
    ones = jnp.ones((L,), jnp.int32)

    def body(j, accs):
        base = j * (UNROLL * L)
        new = []
        for u in range(UNROLL):
            v = d_ref[pl.ds(base + u * L, L)]
            a = accs[u]
            new.append(jnp.where(v >= tvec, a + ones, a))
        return tuple(new)

    accs = lax.fori_loop(
        0, NV // UNROLL, body,
        tuple(jnp.zeros((L,), jnp.int32) for _ in range(UNROLL)),
    )
    tot = accs[0]
    for u in range(1, UNROLL):
        tot = tot + accs[u]
    iota = lax.iota(jnp.int32, L)
    for step in (1, 2, 4, 8):
        tot = tot + tot[jnp.bitwise_xor(iota, jnp.int32(step))]
    return tot[0]


def _kwta_body(x_hbm, out_hbm, d_ref):
    cid = lax.axis_index("c")
    sid = lax.axis_index("s")
    wid = sid * NUM_CORES + cid

    for r in range(ROWS_PER_WORKER):
        row = wid * ROWS_PER_WORKER + r
        pltpu.sync_copy(x_hbm.at[row], d_ref)

        cnt0 = _count_ge(d_ref, jnp.int32(0))
        p0 = jnp.where(cnt0 >= K, jnp.int32(0), jnp.int32(_INT_MIN))

        def bit_body(i, p):
            c = jnp.bitwise_or(p, jnp.int32(1) << (jnp.int32(30) - i))
            cnt = _count_ge(d_ref, c)
            return jnp.where(cnt >= K, c, p)

        p = lax.fori_loop(0, 31, bit_body, p0)

        tvec = _threshold_vec(p)
        zero = jnp.zeros((L,), jnp.float32)

        def mask_body(j, carry):
            base = j * (UNROLL * L)
            for u in range(UNROLL):
                v = d_ref[pl.ds(base + u * L, L)]
                d_ref[pl.ds(base + u * L, L)] = jnp.where(v >= tvec, v, zero)
            return carry

        lax.fori_loop(0, NV // UNROLL, mask_body, jnp.int32(0))

        pltpu.sync_copy(d_ref, out_hbm.at[row])


@jax.jit
def kernel(inputs):
    mesh = plsc.VectorSubcoreMesh(
        core_axis_name="c", subcore_axis_name="s",
        num_cores=NUM_CORES, num_subcores=NUM_SUBCORES,
    )
    f = pl.kernel(
        _kwta_body,
        out_type=jax.ShapeDtypeStruct((ROWS, N), jnp.float32),
        mesh=mesh,
        scratch_types=[pltpu.VMEM((N,), jnp.float32)],
    )
    return f(inputs)
